# Initial kernel scaffold; baseline (speedup 1.0000x reference)
import jax
import jax.numpy as jnp
from jax.experimental import pallas as pl


def kernel(h, edge_index_r0, edge_index_r1, omega, kb, W, self_loop, bias_p, relw):
    raise NotImplementedError("write your pallas kernel here")


# speedup vs baseline: 5.6605x; 5.6605x over previous
import functools

import jax
import jax.numpy as jnp
from jax import lax
from jax.experimental import pallas as pl
from jax.experimental.pallas import tpu as pltpu
from jax.experimental.pallas import tpu_sc as plsc

N = 10000
D = 128
K = 64
OUT = 128
E = 320000
SIGMAS = (1.0, 1.0, 1.0, 1.0)

NC = 2
NS = 16
NW = NC * NS
EPT = E // NW
C = 40
NCHUNK = EPT // C
SB = 50
NSB = NCHUNK // SB
RPS = 624
REM = N - NS * RPS
DW = 8
ZP = 16
ZSTRIDE = N // NS
WP = -(-RPS // C) + 1


def _hk_body(h_ref, om_ref, b_ref, hk0_ref, hk1_ref):
    x = jnp.dot(h_ref[...], om_ref[...], preferred_element_type=jnp.float32)
    y = jnp.float32((2.0 / K) ** 0.5) * jnp.cos(x + b_ref[...])
    hk0_ref[...] = y[:, : 2 * K]
    hk1_ref[...] = y[:, 2 * K :]


def _compute_hk(h, om_cat, b_cat):
    bn = 2000
    grid = (N // bn,)
    return pl.pallas_call(
        _hk_body,
        grid=grid,
        in_specs=[
            pl.BlockSpec((bn, D), lambda i: (i, 0)),
            pl.BlockSpec((D, 4 * K), lambda i: (0, 0)),
            pl.BlockSpec((1, 4 * K), lambda i: (0, 0)),
        ],
        out_specs=[pl.BlockSpec((bn, 2 * K), lambda i: (i, 0))] * 2,
        out_shape=[jax.ShapeDtypeStruct((N, 2 * K), jnp.float32)] * 2,
    )(h, om_cat, b_cat)


def _sc_body(hk0, hk1, row0, col0, row1, col1, zidx, widx, ones_hbm, zg,
             g0p, g1p, d0p, d1p,
             colv, rowv, rows_v, zidxv, widxv, gsh, sem):
    c = lax.axis_index("c")
    s = lax.axis_index("s")
    wid = c * NS + s

    base = s * RPS

    def _zero_accum():
        for z in range(ZP):
            pltpu.sync_copy(rows_v, gsh.at[zidxv.at[z]])

    def _writeout(out):
        def _out(p, o, n):
            pltpu.async_copy(gsh.at[widxv.at[p]], rows_v, sem).wait()
            pltpu.sync_copy(rows_v.at[pl.ds(0, n)], out.at[c, pl.ds(o, n)])

        for p in range(WP - 1):
            _out(p, base + p * C, min(C, RPS - p * C))

        @pl.when(s == 0)
        def _():
            _out(WP - 1, NS * RPS, REM)

    pltpu.sync_copy(zidx.at[s], zidxv)
    pltpu.sync_copy(widx.at[s], widxv)
    pltpu.sync_copy(zg, rows_v)
    _zero_accum()
    plsc.subcore_barrier()

    passes = (
        (hk0, row0, col0, g0p), (hk1, row1, col1, g1p),
        (None, row0, col0, d0p), (None, row1, col1, d1p),
    )
    for i, (hk, rowh, colh, out) in enumerate(passes):
        if hk is not None:
            def _chunk(j, carry):
                pltpu.async_copy(hk.at[colv.at[j]], rows_v, sem).wait()
                pltpu.sync_copy(rows_v, gsh.at[rowv.at[j]], add=True)
                return carry
        else:
            pltpu.sync_copy(ones_hbm, rows_v)

            def _chunk(j, carry):
                pltpu.sync_copy(rows_v, gsh.at[rowv.at[j]], add=True)
                return carry

        for ss in range(NSB):
            if hk is not None:
                pltpu.sync_copy(colh.at[wid, ss], colv)
            pltpu.sync_copy(rowh.at[wid, ss], rowv)
            lax.fori_loop(0, SB, _chunk, 0)
        plsc.subcore_barrier()

        _writeout(out)
        if i != len(passes) - 1:
            plsc.subcore_barrier()
            pltpu.sync_copy(zg, rows_v)
            _zero_accum()
            plsc.subcore_barrier()


def _sc_aggregate(hk0, hk1, row0, col0, row1, col1, zidx, widx, ones_hbm,
                  zg):
    mesh = plsc.VectorSubcoreMesh(core_axis_name="c", subcore_axis_name="s")
    kern = functools.partial(
        pl.kernel,
        mesh=mesh,
        out_type=[jax.ShapeDtypeStruct((NC, N, 2 * K), jnp.float32)] * 4,
        scratch_types=[
            pltpu.VMEM((SB, C), jnp.int32),
            pltpu.VMEM((SB, C), jnp.int32),
            pltpu.VMEM((C, 2 * K), jnp.float32),
            pltpu.VMEM((ZP, C), jnp.int32),
            pltpu.VMEM((WP, C), jnp.int32),
            pltpu.VMEM_SHARED((N, 2 * K), jnp.float32),
            pltpu.SemaphoreType.DMA,
        ],
    )(_sc_body)
    return kern(hk0, hk1, row0, col0, row1, col1, zidx, widx, ones_hbm, zg)


def _mix_body(h_ref, g0_ref, d0_ref, g1_ref, d1_ref, w_ref, sl_ref, b_ref,
              out0_ref, out1_ref):
    g0 = g0_ref[0] + g0_ref[1]
    g1 = g1_ref[0] + g1_ref[1]
    r0 = 1.0 / jnp.maximum(d0_ref[0, :, 0:1] + d0_ref[1, :, 0:1], 1.0)
    r1 = 1.0 / jnp.maximum(d1_ref[0, :, 0:1] + d1_ref[1, :, 0:1], 1.0)
    for si, out_ref in ((0, out0_ref), (1, out1_ref)):
        half = slice(si * K, (si + 1) * K)
        acc = jnp.dot(g0[:, half], w_ref[2 * si + 0],
                      preferred_element_type=jnp.float32) * r0
        acc = acc + jnp.dot(g1[:, half], w_ref[2 * si + 1],
                            preferred_element_type=jnp.float32) * r1
        acc = acc + jnp.dot(h_ref[...], sl_ref[si],
                            preferred_element_type=jnp.float32)
        out_ref[...] = acc + b_ref[si : si + 1, :]


def _mix(h, g0p, d0p, g1p, d1p, w_alpha, self_loop, bias_p):
    bn = 2000
    grid = (N // bn,)
    return pl.pallas_call(
        _mix_body,
        grid=grid,
        in_specs=[
            pl.BlockSpec((bn, D), lambda i: (i, 0)),
            pl.BlockSpec((NC, bn, 2 * K), lambda i: (0, i, 0)),
            pl.BlockSpec((NC, bn, 2 * K), lambda i: (0, i, 0)),
            pl.BlockSpec((NC, bn, 2 * K), lambda i: (0, i, 0)),
            pl.BlockSpec((NC, bn, 2 * K), lambda i: (0, i, 0)),
            pl.BlockSpec((4, K, OUT), lambda i: (0, 0, 0)),
            pl.BlockSpec((2, D, OUT), lambda i: (0, 0, 0)),
            pl.BlockSpec((2, OUT), lambda i: (0, 0)),
        ],
        out_specs=[pl.BlockSpec((bn, OUT), lambda i: (i, 0))] * 2,
        out_shape=[jax.ShapeDtypeStruct((N, OUT), jnp.float32)] * 2,
    )(h, g0p, d0p, g1p, d1p, w_alpha, self_loop, bias_p)


def kernel(h, edge_index_r0, edge_index_r1, omega, kb, W, self_loop, bias_p,
           relw):
    sig = jnp.asarray(SIGMAS, jnp.float32)
    om = omega / sig[:, None, None]
    om_cat = jnp.concatenate([om[0], om[2], om[1], om[3]], axis=1)
    b_cat = jnp.concatenate([kb[0], kb[2], kb[1], kb[3]])[None, :]

    hk0, hk1 = _compute_hk(h, om_cat, b_cat)

    row0 = edge_index_r0[0].reshape(NW, NSB, SB, C)
    col0 = edge_index_r0[1].reshape(NW, NSB, SB, C)
    row1 = edge_index_r1[0].reshape(NW, NSB, SB, C)
    col1 = edge_index_r1[1].reshape(NW, NSB, SB, C)
    ones_hbm = jnp.ones((C, 2 * K), jnp.float32)
    zg = jnp.zeros((C, 2 * K), jnp.float32)
    zidx = jnp.minimum(
        jnp.arange(NS, dtype=jnp.int32)[:, None] * ZSTRIDE
        + jnp.minimum(jnp.arange(ZP * C, dtype=jnp.int32),
                      ZSTRIDE - 1)[None, :],
        N - 1,
    ).reshape(NS, ZP, C)
    widx = jnp.minimum(
        jnp.arange(NS, dtype=jnp.int32)[:, None] * RPS
        + jnp.arange(WP * C, dtype=jnp.int32)[None, :],
        N - 1,
    ).reshape(NS, WP, C)
    widx = widx.at[:, WP - 1, :].set(
        jnp.minimum(NS * RPS + jnp.arange(C, dtype=jnp.int32), N - 1))

    g0p, g1p, d0p, d1p = _sc_aggregate(
        hk0, hk1, row0, col0, row1, col1, zidx, widx, ones_hbm, zg)

    alpha = jax.nn.softmax(relw, axis=1)
    w_alpha = W * alpha.reshape(4)[:, None, None]

    out0, out1 = _mix(h, g0p, d0p, g1p, d1p, w_alpha, self_loop, bias_p)
    return (out0, out1)


# speedup vs baseline: 7.2905x; 1.2880x over previous
import functools

import jax
import jax.numpy as jnp
from jax import lax
from jax.experimental import pallas as pl
from jax.experimental.pallas import tpu as pltpu
from jax.experimental.pallas import tpu_sc as plsc

N = 10000
D = 128
K = 64
OUT = 128
E = 320000
SIGMAS = (1.0, 1.0, 1.0, 1.0)

NC = 2
NS = 16
NW = NC * NS
EPT = E // NW
C = 80
NCHUNK = EPT // C
SB = 25
NSB = NCHUNK // SB
RPS = 624
REM = N - NS * RPS
DW = 8
ZP = 8
ZSTRIDE = N // NS
WP = -(-RPS // C) + 1


def _hk_body(h_ref, om_ref, b_ref, hk0_ref, hk1_ref):
    x = jnp.dot(h_ref[...], om_ref[...], preferred_element_type=jnp.float32)
    y = jnp.float32((2.0 / K) ** 0.5) * jnp.cos(x + b_ref[...])
    hk0_ref[...] = y[:, : 2 * K]
    hk1_ref[...] = y[:, 2 * K :]


def _compute_hk(h, om_cat, b_cat):
    bn = 2000
    grid = (N // bn,)
    return pl.pallas_call(
        _hk_body,
        grid=grid,
        in_specs=[
            pl.BlockSpec((bn, D), lambda i: (i, 0)),
            pl.BlockSpec((D, 4 * K), lambda i: (0, 0)),
            pl.BlockSpec((1, 4 * K), lambda i: (0, 0)),
        ],
        out_specs=[pl.BlockSpec((bn, 2 * K), lambda i: (i, 0))] * 2,
        out_shape=[jax.ShapeDtypeStruct((N, 2 * K), jnp.float32)] * 2,
    )(h, om_cat, b_cat)


def _sc_body(hk0, hk1, row0, col0, row1, col1, zidx, widx, ones_hbm, zg,
             g0p, g1p, d0p, d1p,
             colv, rowv, rows_v, zidxv, widxv, gsh, sem):
    c = lax.axis_index("c")
    s = lax.axis_index("s")
    wid = c * NS + s

    base = s * RPS

    def _zero_accum():
        for z in range(ZP):
            pltpu.sync_copy(rows_v, gsh.at[zidxv.at[z]])

    def _writeout(out):
        def _out(p, o, n):
            pltpu.async_copy(gsh.at[widxv.at[p]], rows_v, sem).wait()
            pltpu.sync_copy(rows_v.at[pl.ds(0, n)], out.at[c, pl.ds(o, n)])

        for p in range(WP - 1):
            _out(p, base + p * C, min(C, RPS - p * C))

        @pl.when(s == 0)
        def _():
            _out(WP - 1, NS * RPS, REM)

    pltpu.sync_copy(zidx.at[s], zidxv)
    pltpu.sync_copy(widx.at[s], widxv)
    pltpu.sync_copy(zg, rows_v)
    _zero_accum()
    plsc.subcore_barrier()

    passes = (
        (hk0, row0, col0, g0p), (hk1, row1, col1, g1p),
        (None, row0, col0, d0p), (None, row1, col1, d1p),
    )
    for i, (hk, rowh, colh, out) in enumerate(passes):
        if hk is not None:
            def _chunk(j, carry):
                pltpu.async_copy(hk.at[colv.at[j]], rows_v, sem).wait()
                pltpu.sync_copy(rows_v, gsh.at[rowv.at[j]], add=True)
                return carry
        else:
            pltpu.sync_copy(ones_hbm, rows_v)

            def _chunk(j, carry):
                pltpu.sync_copy(rows_v, gsh.at[rowv.at[j]], add=True)
                return carry

        for ss in range(NSB):
            if hk is not None:
                pltpu.sync_copy(colh.at[wid, ss], colv)
            pltpu.sync_copy(rowh.at[wid, ss], rowv)
            lax.fori_loop(0, SB, _chunk, 0)
        plsc.subcore_barrier()

        _writeout(out)
        if i != len(passes) - 1:
            plsc.subcore_barrier()
            pltpu.sync_copy(zg, rows_v)
            _zero_accum()
            plsc.subcore_barrier()


def _sc_aggregate(hk0, hk1, row0, col0, row1, col1, zidx, widx, ones_hbm,
                  zg):
    mesh = plsc.VectorSubcoreMesh(core_axis_name="c", subcore_axis_name="s")
    kern = functools.partial(
        pl.kernel,
        mesh=mesh,
        out_type=[jax.ShapeDtypeStruct((NC, N, 2 * K), jnp.float32)] * 4,
        scratch_types=[
            pltpu.VMEM((SB, C), jnp.int32),
            pltpu.VMEM((SB, C), jnp.int32),
            pltpu.VMEM((C, 2 * K), jnp.float32),
            pltpu.VMEM((ZP, C), jnp.int32),
            pltpu.VMEM((WP, C), jnp.int32),
            pltpu.VMEM_SHARED((N, 2 * K), jnp.float32),
            pltpu.SemaphoreType.DMA,
        ],
    )(_sc_body)
    return kern(hk0, hk1, row0, col0, row1, col1, zidx, widx, ones_hbm, zg)


def _mix_body(h_ref, g0_ref, d0_ref, g1_ref, d1_ref, w_ref, sl_ref, b_ref,
              out0_ref, out1_ref):
    g0 = g0_ref[0] + g0_ref[1]
    g1 = g1_ref[0] + g1_ref[1]
    r0 = 1.0 / jnp.maximum(d0_ref[0, :, 0:1] + d0_ref[1, :, 0:1], 1.0)
    r1 = 1.0 / jnp.maximum(d1_ref[0, :, 0:1] + d1_ref[1, :, 0:1], 1.0)
    for si, out_ref in ((0, out0_ref), (1, out1_ref)):
        half = slice(si * K, (si + 1) * K)
        acc = jnp.dot(g0[:, half], w_ref[2 * si + 0],
                      preferred_element_type=jnp.float32) * r0
        acc = acc + jnp.dot(g1[:, half], w_ref[2 * si + 1],
                            preferred_element_type=jnp.float32) * r1
        acc = acc + jnp.dot(h_ref[...], sl_ref[si],
                            preferred_element_type=jnp.float32)
        out_ref[...] = acc + b_ref[si : si + 1, :]


def _mix(h, g0p, d0p, g1p, d1p, w_alpha, self_loop, bias_p):
    bn = 2000
    grid = (N // bn,)
    return pl.pallas_call(
        _mix_body,
        grid=grid,
        in_specs=[
            pl.BlockSpec((bn, D), lambda i: (i, 0)),
            pl.BlockSpec((NC, bn, 2 * K), lambda i: (0, i, 0)),
            pl.BlockSpec((NC, bn, 2 * K), lambda i: (0, i, 0)),
            pl.BlockSpec((NC, bn, 2 * K), lambda i: (0, i, 0)),
            pl.BlockSpec((NC, bn, 2 * K), lambda i: (0, i, 0)),
            pl.BlockSpec((4, K, OUT), lambda i: (0, 0, 0)),
            pl.BlockSpec((2, D, OUT), lambda i: (0, 0, 0)),
            pl.BlockSpec((2, OUT), lambda i: (0, 0)),
        ],
        out_specs=[pl.BlockSpec((bn, OUT), lambda i: (i, 0))] * 2,
        out_shape=[jax.ShapeDtypeStruct((N, OUT), jnp.float32)] * 2,
    )(h, g0p, d0p, g1p, d1p, w_alpha, self_loop, bias_p)


def kernel(h, edge_index_r0, edge_index_r1, omega, kb, W, self_loop, bias_p,
           relw):
    sig = jnp.asarray(SIGMAS, jnp.float32)
    om = omega / sig[:, None, None]
    om_cat = jnp.concatenate([om[0], om[2], om[1], om[3]], axis=1)
    b_cat = jnp.concatenate([kb[0], kb[2], kb[1], kb[3]])[None, :]

    hk0, hk1 = _compute_hk(h, om_cat, b_cat)

    row0 = edge_index_r0[0].reshape(NW, NSB, SB, C)
    col0 = edge_index_r0[1].reshape(NW, NSB, SB, C)
    row1 = edge_index_r1[0].reshape(NW, NSB, SB, C)
    col1 = edge_index_r1[1].reshape(NW, NSB, SB, C)
    ones_hbm = jnp.ones((C, 2 * K), jnp.float32)
    zg = jnp.zeros((C, 2 * K), jnp.float32)
    zidx = jnp.minimum(
        jnp.arange(NS, dtype=jnp.int32)[:, None] * ZSTRIDE
        + jnp.minimum(jnp.arange(ZP * C, dtype=jnp.int32),
                      ZSTRIDE - 1)[None, :],
        N - 1,
    ).reshape(NS, ZP, C)
    widx = jnp.minimum(
        jnp.arange(NS, dtype=jnp.int32)[:, None] * RPS
        + jnp.arange(WP * C, dtype=jnp.int32)[None, :],
        N - 1,
    ).reshape(NS, WP, C)
    widx = widx.at[:, WP - 1, :].set(
        jnp.minimum(NS * RPS + jnp.arange(C, dtype=jnp.int32), N - 1))

    g0p, g1p, d0p, d1p = _sc_aggregate(
        hk0, hk1, row0, col0, row1, col1, zidx, widx, ones_hbm, zg)

    alpha = jax.nn.softmax(relw, axis=1)
    w_alpha = W * alpha.reshape(4)[:, None, None]

    out0, out1 = _mix(h, g0p, d0p, g1p, d1p, w_alpha, self_loop, bias_p)
    return (out0, out1)


# speedup vs baseline: 7.8749x; 1.0802x over previous
import functools

import jax
import jax.numpy as jnp
from jax import lax
from jax.experimental import pallas as pl
from jax.experimental.pallas import tpu as pltpu
from jax.experimental.pallas import tpu_sc as plsc

N = 10000
D = 128
K = 64
OUT = 128
E = 320000
SIGMAS = (1.0, 1.0, 1.0, 1.0)

NC = 2
NS = 16
NW = NC * NS
EPT = E // NW
C = 80
NCHUNK = EPT // C
SB = 25
NSB = NCHUNK // SB
RPS = 624
REM = N - NS * RPS
DW = 8
ZP = 8
ZSTRIDE = N // NS
WP = -(-RPS // C) + 1


def _hk_body(h_ref, om_ref, b_ref, hk0_ref, hk1_ref):
    x = jnp.dot(h_ref[...], om_ref[...], preferred_element_type=jnp.float32)
    y = jnp.float32((2.0 / K) ** 0.5) * jnp.cos(x + b_ref[...])
    hk0_ref[...] = y[:, : 2 * K]
    hk1_ref[...] = y[:, 2 * K :]


def _compute_hk(h, om_cat, b_cat):
    bn = 2000
    grid = (N // bn,)
    return pl.pallas_call(
        _hk_body,
        grid=grid,
        in_specs=[
            pl.BlockSpec((bn, D), lambda i: (i, 0)),
            pl.BlockSpec((D, 4 * K), lambda i: (0, 0)),
            pl.BlockSpec((1, 4 * K), lambda i: (0, 0)),
        ],
        out_specs=[pl.BlockSpec((bn, 2 * K), lambda i: (i, 0))] * 2,
        out_shape=[jax.ShapeDtypeStruct((N, 2 * K), jnp.float32)] * 2,
    )(h, om_cat, b_cat)


def _sc_body(hk0, hk1, row0, col0, row1, col1, zidx, widx, ones_hbm, zg,
             g0p, g1p, d0p, d1p,
             colv, rowv, rows_v, rows_v2, zidxv, widxv, gsh, sem, sem2):
    c = lax.axis_index("c")
    s = lax.axis_index("s")
    wid = c * NS + s

    base = s * RPS

    def _zero_accum():
        for z in range(ZP):
            pltpu.sync_copy(rows_v, gsh.at[zidxv.at[z]])

    def _writeout(out):
        def _out(p, o, n):
            pltpu.async_copy(gsh.at[widxv.at[p]], rows_v, sem).wait()
            pltpu.sync_copy(rows_v.at[pl.ds(0, n)], out.at[c, pl.ds(o, n)])

        for p in range(WP - 1):
            _out(p, base + p * C, min(C, RPS - p * C))

        @pl.when(s == 0)
        def _():
            _out(WP - 1, NS * RPS, REM)

    pltpu.sync_copy(zidx.at[s], zidxv)
    pltpu.sync_copy(widx.at[s], widxv)
    pltpu.sync_copy(zg, rows_v)
    _zero_accum()
    plsc.subcore_barrier()

    passes = (
        (hk0, row0, col0, g0p), (hk1, row1, col1, g1p),
        (None, row0, col0, d0p), (None, row1, col1, d1p),
    )
    for i, (hk, rowh, colh, out) in enumerate(passes):
        if hk is not None:
            def _pair(j2, carry):
                ja = 2 * j2
                jb = ja + 1
                pltpu.async_copy(hk.at[colv.at[ja]], rows_v, sem).wait()
                sa = pltpu.async_copy(rows_v, gsh.at[rowv.at[ja]], sem2,
                                      add=True)
                pltpu.async_copy(hk.at[colv.at[jb]], rows_v2, sem).wait()
                sa.wait()
                pltpu.async_copy(rows_v2, gsh.at[rowv.at[jb]], sem2,
                                 add=True).wait()
                return carry

            def _tail():
                for j in range(2 * (SB // 2), SB):
                    pltpu.async_copy(hk.at[colv.at[j]], rows_v, sem).wait()
                    pltpu.sync_copy(rows_v, gsh.at[rowv.at[j]], add=True)
        else:
            pltpu.sync_copy(ones_hbm, rows_v)

            def _chunk(j, carry):
                pltpu.async_copy(rows_v, gsh.at[rowv.at[j]], sem2, add=True)
                return carry

        for ss in range(NSB):
            if hk is not None:
                pltpu.sync_copy(colh.at[wid, ss], colv)
                pltpu.sync_copy(rowh.at[wid, ss], rowv)
                lax.fori_loop(0, SB // 2, _pair, 0)
                _tail()
            else:
                pltpu.sync_copy(rowh.at[wid, ss], rowv)
                lax.fori_loop(0, SB, _chunk, 0)
                for _ in range(SB):
                    pltpu.make_async_copy(
                        rows_v, gsh.at[rowv.at[0]], sem2).wait()
        plsc.subcore_barrier()

        _writeout(out)
        if i != len(passes) - 1:
            plsc.subcore_barrier()
            pltpu.sync_copy(zg, rows_v)
            _zero_accum()
            plsc.subcore_barrier()


def _sc_aggregate(hk0, hk1, row0, col0, row1, col1, zidx, widx, ones_hbm,
                  zg):
    mesh = plsc.VectorSubcoreMesh(core_axis_name="c", subcore_axis_name="s")
    kern = functools.partial(
        pl.kernel,
        mesh=mesh,
        out_type=[jax.ShapeDtypeStruct((NC, N, 2 * K), jnp.float32)] * 4,
        scratch_types=[
            pltpu.VMEM((SB, C), jnp.int32),
            pltpu.VMEM((SB, C), jnp.int32),
            pltpu.VMEM((C, 2 * K), jnp.float32),
            pltpu.VMEM((C, 2 * K), jnp.float32),
            pltpu.VMEM((ZP, C), jnp.int32),
            pltpu.VMEM((WP, C), jnp.int32),
            pltpu.VMEM_SHARED((N, 2 * K), jnp.float32),
            pltpu.SemaphoreType.DMA,
            pltpu.SemaphoreType.DMA,
        ],
    )(_sc_body)
    return kern(hk0, hk1, row0, col0, row1, col1, zidx, widx, ones_hbm, zg)


def _mix_body(h_ref, g0_ref, d0_ref, g1_ref, d1_ref, w_ref, sl_ref, b_ref,
              out0_ref, out1_ref):
    g0 = g0_ref[0] + g0_ref[1]
    g1 = g1_ref[0] + g1_ref[1]
    r0 = 1.0 / jnp.maximum(d0_ref[0, :, 0:1] + d0_ref[1, :, 0:1], 1.0)
    r1 = 1.0 / jnp.maximum(d1_ref[0, :, 0:1] + d1_ref[1, :, 0:1], 1.0)
    for si, out_ref in ((0, out0_ref), (1, out1_ref)):
        half = slice(si * K, (si + 1) * K)
        acc = jnp.dot(g0[:, half], w_ref[2 * si + 0],
                      preferred_element_type=jnp.float32) * r0
        acc = acc + jnp.dot(g1[:, half], w_ref[2 * si + 1],
                            preferred_element_type=jnp.float32) * r1
        acc = acc + jnp.dot(h_ref[...], sl_ref[si],
                            preferred_element_type=jnp.float32)
        out_ref[...] = acc + b_ref[si : si + 1, :]


def _mix(h, g0p, d0p, g1p, d1p, w_alpha, self_loop, bias_p):
    bn = 2000
    grid = (N // bn,)
    return pl.pallas_call(
        _mix_body,
        grid=grid,
        in_specs=[
            pl.BlockSpec((bn, D), lambda i: (i, 0)),
            pl.BlockSpec((NC, bn, 2 * K), lambda i: (0, i, 0)),
            pl.BlockSpec((NC, bn, 2 * K), lambda i: (0, i, 0)),
            pl.BlockSpec((NC, bn, 2 * K), lambda i: (0, i, 0)),
            pl.BlockSpec((NC, bn, 2 * K), lambda i: (0, i, 0)),
            pl.BlockSpec((4, K, OUT), lambda i: (0, 0, 0)),
            pl.BlockSpec((2, D, OUT), lambda i: (0, 0, 0)),
            pl.BlockSpec((2, OUT), lambda i: (0, 0)),
        ],
        out_specs=[pl.BlockSpec((bn, OUT), lambda i: (i, 0))] * 2,
        out_shape=[jax.ShapeDtypeStruct((N, OUT), jnp.float32)] * 2,
    )(h, g0p, d0p, g1p, d1p, w_alpha, self_loop, bias_p)


def kernel(h, edge_index_r0, edge_index_r1, omega, kb, W, self_loop, bias_p,
           relw):
    sig = jnp.asarray(SIGMAS, jnp.float32)
    om = omega / sig[:, None, None]
    om_cat = jnp.concatenate([om[0], om[2], om[1], om[3]], axis=1)
    b_cat = jnp.concatenate([kb[0], kb[2], kb[1], kb[3]])[None, :]

    hk0, hk1 = _compute_hk(h, om_cat, b_cat)

    row0 = edge_index_r0[0].reshape(NW, NSB, SB, C)
    col0 = edge_index_r0[1].reshape(NW, NSB, SB, C)
    row1 = edge_index_r1[0].reshape(NW, NSB, SB, C)
    col1 = edge_index_r1[1].reshape(NW, NSB, SB, C)
    ones_hbm = jnp.ones((C, 2 * K), jnp.float32)
    zg = jnp.zeros((C, 2 * K), jnp.float32)
    zidx = jnp.minimum(
        jnp.arange(NS, dtype=jnp.int32)[:, None] * ZSTRIDE
        + jnp.minimum(jnp.arange(ZP * C, dtype=jnp.int32),
                      ZSTRIDE - 1)[None, :],
        N - 1,
    ).reshape(NS, ZP, C)
    widx = jnp.minimum(
        jnp.arange(NS, dtype=jnp.int32)[:, None] * RPS
        + jnp.arange(WP * C, dtype=jnp.int32)[None, :],
        N - 1,
    ).reshape(NS, WP, C)
    widx = widx.at[:, WP - 1, :].set(
        jnp.minimum(NS * RPS + jnp.arange(C, dtype=jnp.int32), N - 1))

    g0p, g1p, d0p, d1p = _sc_aggregate(
        hk0, hk1, row0, col0, row1, col1, zidx, widx, ones_hbm, zg)

    alpha = jax.nn.softmax(relw, axis=1)
    w_alpha = W * alpha.reshape(4)[:, None, None]

    out0, out1 = _mix(h, g0p, d0p, g1p, d1p, w_alpha, self_loop, bias_p)
    return (out0, out1)
